# Initial kernel scaffold; baseline (speedup 1.0000x reference)
import jax
import jax.numpy as jnp
from jax.experimental import pallas as pl


def kernel(pc, node_tag, W1, W2, W3, W4, W5, W6, W7, W8, W9, W10, W11, W12, W13, Wr, br):
    raise NotImplementedError("write your pallas kernel here")


# speedup vs baseline: 1.0016x; 1.0016x over previous
import functools

import jax
import jax.numpy as jnp
from jax.experimental import pallas as pl
from jax.experimental.pallas import tpu as pltpu

K = 20
EPS = 1e-5


def _lrelu(x):
    return jnp.where(x >= 0, x, 0.2 * x)


def _inorm(x):
    mu = jnp.mean(x, axis=1, keepdims=True)
    var = jnp.var(x, axis=1, keepdims=True)
    return (x - mu) / jnp.sqrt(var + EPS)


def _knn_idx(x, batch, k):
    n = x.shape[0]
    sq = jnp.sum(x * x, axis=1)
    d2 = sq[:, None] + sq[None, :] - 2.0 * (x @ x.T)
    mask = (batch[:, None] != batch[None, :]) | jnp.eye(n, dtype=bool)
    d2 = jnp.where(mask, jnp.inf, d2)
    _, nbr = jax.lax.top_k(-d2, k)
    return nbr.reshape(-1)


def _edge_block(x, idx, wa, wb, n):
    c = x.shape[1]
    f = x[idx].reshape(c, -1)
    f = _lrelu(_inorm(wa @ f))
    f = _lrelu(_inorm(wb @ f))
    vals = f.T
    out = jnp.full((n, f.shape[0]), -jnp.inf, dtype=f.dtype).at[idx].max(vals)
    return jnp.where(jnp.isneginf(out), 0.0, out)


def _tail_body(x1, x2, x3, x4, w9, w10, w11, w12, w13, wr, br, out):
    def norm_act(a):
        mu = jnp.mean(a, axis=1, keepdims=True)
        var = jnp.mean(a * a, axis=1, keepdims=True) - mu * mu
        y = (a - mu) * jax.lax.rsqrt(var + EPS)
        return jnp.where(y >= 0, y, 0.2 * y)

    dimn = (((1,), (1,)), ((), ()))
    xc = jnp.concatenate([x1[...], x2[...], x3[...], x4[...]], axis=1)
    g = norm_act(jax.lax.dot_general(w9[...], xc, dimn,
                                     preferred_element_type=jnp.float32))
    g = jnp.broadcast_to(jnp.max(g, axis=1, keepdims=True), g.shape)

    y4 = norm_act(jnp.dot(w10[...],
                          jnp.concatenate([g, x4[...].T], axis=0),
                          preferred_element_type=jnp.float32))
    y3 = norm_act(jnp.dot(w11[...],
                          jnp.concatenate([y4, x3[...].T], axis=0),
                          preferred_element_type=jnp.float32))
    y2 = norm_act(jnp.dot(w12[...],
                          jnp.concatenate([y3, x2[...].T], axis=0),
                          preferred_element_type=jnp.float32))
    y1 = norm_act(jnp.dot(w13[...],
                          jnp.concatenate([y2, x1[...].T], axis=0),
                          preferred_element_type=jnp.float32))
    code_t = jax.lax.dot_general(y1, wr[...], (((0,), (1,)), ((), ())),
                                 preferred_element_type=jnp.float32)
    out[...] = code_t + br[...][None, :]


def _tail(x1, x2, x3, x4, w9, w10, w11, w12, w13, wr, br):
    n = x1.shape[0]
    c = wr.shape[0]
    return pl.pallas_call(
        _tail_body,
        out_shape=jax.ShapeDtypeStruct((n, c), jnp.float32),
    )(x1, x2, x3, x4, w9, w10, w11, w12, w13, wr, br)


def kernel(pc, node_tag, W1, W2, W3, W4, W5, W6, W7, W8, W9, W10, W11, W12,
           W13, Wr, br):
    bs, n_nodes, _ = pc.shape
    n = bs * n_nodes
    batch = node_tag.reshape(-1)
    x0 = pc.reshape(-1, 3)
    idx = _knn_idx(x0, batch, K)
    x1 = _edge_block(x0, idx, W1, W2, n)
    idx = _knn_idx(x1, batch, K)
    x2 = _edge_block(x1, idx, W3, W4, n) + x1
    idx = _knn_idx(x2, batch, K)
    x3 = _edge_block(x2, idx, W5, W6, n) + x2
    idx = _knn_idx(x3, batch, K)
    x4 = _edge_block(x3, idx, W7, W8, n) + x3
    code = _tail(x1, x2, x3, x4, W9, W10, W11, W12, W13, Wr, br)
    codes = code.reshape(bs, n_nodes, -1)
    return (codes, pc)


# speedup vs baseline: 3.8261x; 3.8202x over previous
import functools

import jax
import jax.numpy as jnp
from jax.experimental import pallas as pl
from jax.experimental.pallas import tpu as pltpu

K = 20
EPS = 1e-5


def _lrelu(x):
    return jnp.where(x >= 0, x, 0.2 * x)


def _inorm(x):
    mu = jnp.mean(x, axis=1, keepdims=True)
    var = jnp.var(x, axis=1, keepdims=True)
    return (x - mu) / jnp.sqrt(var + EPS)


_MASKED = 3.0e38


def _topk_body(d2_ref, out_ref, *, k):
    d2 = d2_ref[...]
    r, n = d2.shape
    d2 = jnp.where(d2 >= _MASKED, _MASKED, d2)
    colid = jax.lax.broadcasted_iota(jnp.int32, (r, n), 1)
    cols = []
    for _ in range(k):
        v = jnp.min(d2, axis=1, keepdims=True)
        cand = jnp.where(d2 == v, colid, jnp.int32(2 ** 30))
        i = jnp.min(cand, axis=1, keepdims=True)
        cols.append(i)
        d2 = jnp.where(colid == i, jnp.inf, d2)
    pad = jnp.zeros((r, 128 - k), dtype=jnp.int32)
    out_ref[...] = jnp.concatenate(cols + [pad], axis=1)


def _knn_idx(x, batch, k):
    n = x.shape[0]
    sq = jnp.sum(x * x, axis=1)
    d2 = sq[:, None] + sq[None, :] - 2.0 * (x @ x.T)
    mask = (batch[:, None] != batch[None, :]) | jnp.eye(n, dtype=bool)
    d2 = jnp.where(mask, jnp.inf, d2)
    R = 512
    nbr = pl.pallas_call(
        functools.partial(_topk_body, k=k),
        grid=(n // R,),
        in_specs=[pl.BlockSpec((R, n), lambda b: (b, 0))],
        out_specs=pl.BlockSpec((R, 128), lambda b: (b, 0)),
        out_shape=jax.ShapeDtypeStruct((n, 128), jnp.int32),
    )(d2)
    return nbr[:, :k].reshape(-1)


def _edge_block(x, idx, wa, wb, n):
    c = x.shape[1]
    f = x[idx].reshape(c, -1)
    f = _lrelu(_inorm(wa @ f))
    f = _lrelu(_inorm(wb @ f))
    vals = f.T
    out = jnp.full((n, f.shape[0]), -jnp.inf, dtype=f.dtype).at[idx].max(vals)
    return jnp.where(jnp.isneginf(out), 0.0, out)


def _tail_body(x1, x2, x3, x4, w9, w10, w11, w12, w13, wr, br, out):
    def norm_act(a):
        mu = jnp.mean(a, axis=1, keepdims=True)
        var = jnp.mean(a * a, axis=1, keepdims=True) - mu * mu
        y = (a - mu) * jax.lax.rsqrt(var + EPS)
        return jnp.where(y >= 0, y, 0.2 * y)

    dimn = (((1,), (1,)), ((), ()))
    xc = jnp.concatenate([x1[...], x2[...], x3[...], x4[...]], axis=1)
    g = norm_act(jax.lax.dot_general(w9[...], xc, dimn,
                                     preferred_element_type=jnp.float32))
    g = jnp.broadcast_to(jnp.max(g, axis=1, keepdims=True), g.shape)

    y4 = norm_act(jnp.dot(w10[...],
                          jnp.concatenate([g, x4[...].T], axis=0),
                          preferred_element_type=jnp.float32))
    y3 = norm_act(jnp.dot(w11[...],
                          jnp.concatenate([y4, x3[...].T], axis=0),
                          preferred_element_type=jnp.float32))
    y2 = norm_act(jnp.dot(w12[...],
                          jnp.concatenate([y3, x2[...].T], axis=0),
                          preferred_element_type=jnp.float32))
    y1 = norm_act(jnp.dot(w13[...],
                          jnp.concatenate([y2, x1[...].T], axis=0),
                          preferred_element_type=jnp.float32))
    code_t = jax.lax.dot_general(y1, wr[...], (((0,), (1,)), ((), ())),
                                 preferred_element_type=jnp.float32)
    out[...] = code_t + br[...][None, :]


def _tail(x1, x2, x3, x4, w9, w10, w11, w12, w13, wr, br):
    n = x1.shape[0]
    c = wr.shape[0]
    return pl.pallas_call(
        _tail_body,
        out_shape=jax.ShapeDtypeStruct((n, c), jnp.float32),
    )(x1, x2, x3, x4, w9, w10, w11, w12, w13, wr, br)


def kernel(pc, node_tag, W1, W2, W3, W4, W5, W6, W7, W8, W9, W10, W11, W12,
           W13, Wr, br):
    bs, n_nodes, _ = pc.shape
    n = bs * n_nodes
    batch = node_tag.reshape(-1)
    x0 = pc.reshape(-1, 3)
    idx = _knn_idx(x0, batch, K)
    x1 = _edge_block(x0, idx, W1, W2, n)
    idx = _knn_idx(x1, batch, K)
    x2 = _edge_block(x1, idx, W3, W4, n) + x1
    idx = _knn_idx(x2, batch, K)
    x3 = _edge_block(x2, idx, W5, W6, n) + x2
    idx = _knn_idx(x3, batch, K)
    x4 = _edge_block(x3, idx, W7, W8, n) + x3
    code = _tail(x1, x2, x3, x4, W9, W10, W11, W12, W13, Wr, br)
    codes = code.reshape(bs, n_nodes, -1)
    return (codes, pc)
